# baseline (speedup 1.0000x reference)
import functools

import jax
import jax.numpy as jnp
from jax.experimental import pallas as pl

_N = 10000
_GRAPHS = 8


def _cheb_mm(txs, W, b):
    K = W.shape[0]
    n, f_in = txs[0].shape
    f_out = W.shape[2]
    blk = 1000
    assert n % blk == 0

    def body(*refs):
        tx_refs = refs[:K]
        w_ref, b_ref, o_ref = refs[K], refs[K + 1], refs[K + 2]
        acc = jnp.dot(tx_refs[0][...], w_ref[0], preferred_element_type=jnp.float32)
        for k in range(1, K):
            acc = acc + jnp.dot(tx_refs[k][...], w_ref[k],
                                preferred_element_type=jnp.float32)
        o_ref[...] = jnp.maximum(acc + b_ref[...], 0.0)

    return pl.pallas_call(
        body,
        grid=(n // blk,),
        in_specs=[pl.BlockSpec((blk, f_in), lambda i: (i, 0))] * K
        + [
            pl.BlockSpec((K, f_in, f_out), lambda i: (0, 0, 0)),
            pl.BlockSpec((1, f_out), lambda i: (0, 0)),
        ],
        out_specs=pl.BlockSpec((blk, f_out), lambda i: (i, 0)),
        out_shape=jax.ShapeDtypeStruct((n, f_out), jnp.float32),
    )(*txs, W, b.reshape(1, -1))


def _prop(h, src, dst, w):
    return jax.ops.segment_sum(w[:, None] * h[src], dst, num_segments=_N)


def _cheb_layer(x, src, dst, w, W, b):
    K = W.shape[0]
    tx0 = x
    tx1 = _prop(x, src, dst, w)
    txs = [tx0, tx1]
    for _ in range(2, K):
        tx2 = 2.0 * _prop(txs[-1], src, dst, w) - txs[-2]
        txs.append(tx2)
    return _cheb_mm(txs, W, b)


def kernel(x, edge_index, edge_weight, batch, W1, b1, W2, b2,
           gw1, gb1, gw2, gb2, gw3, gb3):
    src, dst = edge_index[0], edge_index[1]
    deg = jax.ops.segment_sum(edge_weight, src, num_segments=_N)
    dinv = jnp.where(deg > 0, jax.lax.rsqrt(jnp.where(deg > 0, deg, 1.0)), 0.0)
    w = -(dinv[src] * edge_weight * dinv[dst])

    h = _cheb_layer(x, src, dst, w, W1, b1)
    h = _cheb_layer(h, src, dst, w, W2, b2)

    a = jax.nn.relu(h @ gw1 + gb1)
    a = jax.nn.relu(a @ gw2 + gb2)
    gate = jnp.tanh(a @ gw3 + gb3)[:, 0]
    seg_max = jax.ops.segment_max(gate, batch, num_segments=_GRAPHS)
    seg_max = jnp.where(jnp.isfinite(seg_max), seg_max, 0.0)
    g = jnp.exp(gate - seg_max[batch])
    denom = jax.ops.segment_sum(g, batch, num_segments=_GRAPHS)
    alpha = g / jnp.where(denom > 0, denom, 1.0)[batch]
    return jax.ops.segment_sum(alpha[:, None] * h, batch, num_segments=_GRAPHS)


# speedup vs baseline: 1.0343x; 1.0343x over previous
import functools

import jax
import jax.numpy as jnp
from jax import lax
from jax.experimental import pallas as pl
from jax.experimental.pallas import tpu as pltpu
from jax.experimental.pallas import tpu_sc as plsc

_N = 10000
_E = 160000
_F = 512
_GRAPHS = 8

_RPB = 160
_NSTRIPE = 63
_B = 64
_BLK = 1000

_MESH = plsc.VectorSubcoreMesh(core_axis_name="c", subcore_axis_name="s")


@functools.partial(
    pl.kernel,
    mesh=_MESH,
    out_type=jax.ShapeDtypeStruct((_N, _F), jnp.float32),
    scratch_types=[
        pltpu.VMEM((_RPB, _F), jnp.float32),
        pltpu.VMEM((_B, _F), jnp.float32),
        pltpu.VMEM((256,), jnp.int32),
        pltpu.VMEM((32, 16), jnp.int32),
        pltpu.SemaphoreType.DMA,
    ],
)
def _sprop(u_hbm, pk_hbm, bnd_hbm, out_hbm, acc, rows, ebuf, bv, sem):
    cid = lax.axis_index("c")
    sid = lax.axis_index("s")
    wid = sid * 2 + cid
    lanes = jax.lax.iota(jnp.int32, 16)
    z = jnp.zeros((16,), jnp.float32)

    pltpu.sync_copy(bnd_hbm, bv)
    brow = bv[wid, pl.ds(0, 16)]

    def do_stripe(kr, nrows, subsizes, e0, e1):
        row_lo = kr * _RPB

        def zr(r, _):
            for kk in range(_F // 16):
                acc[r, pl.ds(kk * 16, 16)] = z
            return 0

        lax.fori_loop(0, nrows, zr, 0)

        eb0 = lax.div(e0, _B)
        nb = lax.div(e1 + (_B - 1), _B) - eb0

        def bbody(j, _):
            base = (eb0 + j) * _B
            pltpu.sync_copy(pk_hbm.at[pl.ds(base * 3, 192)],
                            ebuf.at[pl.ds(0, 192)])
            pltpu.async_copy(u_hbm.at[ebuf.at[pl.ds(0, _B)]], rows,
                             sem).wait()
            for t in range(_B // 16):
                off = t * 16
                pos = base + off + lanes
                m = (pos >= e0) & (pos < e1)
                dv = ebuf[pl.ds(64 + off, 16)]
                wv = lax.bitcast_convert_type(ebuf[pl.ds(128 + off, 16)],
                                              jnp.float32)
                dl = jnp.clip(dv - row_lo, 0, nrows - 1)
                wv = jnp.where(m, wv, 0.0)
                ebuf[pl.ds(64 + off, 16)] = dl
                ebuf[pl.ds(128 + off, 16)] = lax.bitcast_convert_type(
                    wv, jnp.int32)

            def rbody(r, _):
                dl = ebuf[pl.ds(64 + r, 16)][0]
                w0 = lax.bitcast_convert_type(ebuf[pl.ds(128 + r, 16)],
                                              jnp.float32)[0]
                wb = jnp.broadcast_to(w0, (16,))
                for kk in range(_F // 16):
                    sl = pl.ds(kk * 16, 16)
                    acc[dl, sl] = acc[dl, sl] + wb * rows[r, sl]
                return 0

            lax.fori_loop(0, _B, rbody, 0)
            return 0

        lax.fori_loop(0, nb, bbody, 0)

        off = 0
        for sz in subsizes:
            pltpu.sync_copy(acc.at[pl.ds(off, sz)],
                            out_hbm.at[pl.ds(row_lo + off, sz)])
            off += sz

    @pl.when(wid < 31)
    def _():
        do_stripe(2 * wid, _RPB, (64, 64, 32), brow[0], brow[1])
        do_stripe(2 * wid + 1, _RPB, (64, 64, 32), brow[2], brow[3])

    @pl.when(wid == 31)
    def _():
        do_stripe(62, _N - 62 * _RPB, (64, 16), brow[0], brow[1])


def _ew_call(body, n_out, *arrays):
    n = arrays[0].shape[0]
    outs = tuple(jax.ShapeDtypeStruct((n, _F), jnp.float32)
                 for _ in range(n_out))
    out_specs = tuple(pl.BlockSpec((_BLK, _F), lambda i: (i, 0))
                      for _ in range(n_out))
    if n_out == 1:
        outs = outs[0]
        out_specs = out_specs[0]
    return pl.pallas_call(
        body,
        grid=(n // _BLK,),
        in_specs=[pl.BlockSpec((_BLK, _F), lambda i: (i, 0))] * len(arrays),
        out_specs=out_specs,
        out_shape=outs,
    )(*arrays)


def _mul(a, b):
    def body(a_ref, b_ref, o_ref):
        o_ref[...] = a_ref[...] * b_ref[...]

    return _ew_call(body, 1, a, b)


def _mid_first(v, dinv):
    def body(v_ref, d_ref, tx_ref, u_ref):
        tx = -(d_ref[...] * v_ref[...])
        tx_ref[...] = tx
        u_ref[...] = d_ref[...] * tx

    return _ew_call(body, 2, v, dinv)


def _mid_recur(v, tprev, dinv):
    def body(v_ref, t_ref, d_ref, tx_ref, u_ref):
        tx = -2.0 * (d_ref[...] * v_ref[...]) - t_ref[...]
        tx_ref[...] = tx
        u_ref[...] = d_ref[...] * tx

    return _ew_call(body, 2, v, tprev, dinv)


def _cheb_mm(txs, W, b):
    K = W.shape[0]
    n, f_in = txs[0].shape
    f_out = W.shape[2]

    def body(*refs):
        tx_refs = refs[:K]
        w_ref, b_ref, o_ref = refs[K], refs[K + 1], refs[K + 2]
        acc = jnp.dot(tx_refs[0][...], w_ref[0], preferred_element_type=jnp.float32)
        for k in range(1, K):
            acc = acc + jnp.dot(tx_refs[k][...], w_ref[k],
                                preferred_element_type=jnp.float32)
        o_ref[...] = jnp.maximum(acc + b_ref[...], 0.0)

    return pl.pallas_call(
        body,
        grid=(n // _BLK,),
        in_specs=[pl.BlockSpec((_BLK, f_in), lambda i: (i, 0))] * K
        + [
            pl.BlockSpec((K, f_in, f_out), lambda i: (0, 0, 0)),
            pl.BlockSpec((1, f_out), lambda i: (0, 0)),
        ],
        out_specs=pl.BlockSpec((_BLK, f_out), lambda i: (i, 0)),
        out_shape=jax.ShapeDtypeStruct((n, f_out), jnp.float32),
    )(*txs, W, b.reshape(1, -1))


def _attention(h, batch, gw1, gb1, gw2, gb2, gw3, gb3):
    n, f = h.shape

    def body(h_ref, b_ref, w1_ref, b1_ref, w2_ref, b2_ref, w3_ref, b3_ref,
             o_ref):
        hv = h_ref[...]
        a = jnp.maximum(jnp.dot(hv, w1_ref[...],
                                preferred_element_type=jnp.float32)
                        + b1_ref[...], 0.0)
        a = jnp.maximum(jnp.dot(a, w2_ref[...],
                                preferred_element_type=jnp.float32)
                        + b2_ref[...], 0.0)
        gate = jnp.tanh(jnp.dot(a, w3_ref[...],
                                preferred_element_type=jnp.float32)
                        + b3_ref[...])
        seg = jax.lax.broadcasted_iota(jnp.int32, (1, _GRAPHS), 1)
        onehot = (b_ref[...] == seg).astype(jnp.float32)
        neg = jnp.float32(-1e30)
        masked = jnp.where(onehot > 0, gate, neg)
        seg_max = jnp.max(masked, axis=0, keepdims=True)
        seg_max = jnp.where(seg_max > neg * 0.5, seg_max, 0.0)
        gmax = jnp.sum(onehot * seg_max, axis=1, keepdims=True)
        g = jnp.exp(gate - gmax)
        denom = jnp.sum(onehot * g, axis=0, keepdims=True)
        safe = jnp.where(denom > 0, denom, 1.0)
        dn = jnp.sum(onehot * safe, axis=1, keepdims=True)
        alpha = g / dn
        weights = onehot * alpha
        o_ref[...] = jax.lax.dot_general(
            weights, hv, (((0,), (0,)), ((), ())),
            preferred_element_type=jnp.float32)

    return pl.pallas_call(
        body,
        out_shape=jax.ShapeDtypeStruct((_GRAPHS, f), jnp.float32),
    )(h, batch.reshape(n, 1), gw1, gb1.reshape(1, -1), gw2,
      gb2.reshape(1, -1), gw3, gb3.reshape(1, -1))


def kernel(x, edge_index, edge_weight, batch, W1, b1, W2, b2,
           gw1, gb1, gw2, gb2, gw3, gb3):
    src = edge_index[0].astype(jnp.int32)
    dst = edge_index[1].astype(jnp.int32)
    ew = edge_weight.astype(jnp.float32)

    perm = jnp.argsort(dst)
    ss_ = src[perm]
    ds_ = dst[perm]
    ws_ = ew[perm]
    packed = jnp.stack(
        [ss_.reshape(-1, _B), ds_.reshape(-1, _B),
         lax.bitcast_convert_type(ws_, jnp.int32).reshape(-1, _B)],
        axis=1).reshape(-1)
    bnds = jnp.searchsorted(
        ds_, jnp.arange(0, 64 * _RPB, _RPB, dtype=jnp.int32)).astype(jnp.int32)
    iw = jnp.arange(31, dtype=jnp.int32)
    rows4 = jnp.stack([bnds[2 * iw], bnds[2 * iw + 1],
                       bnds[2 * iw + 1], bnds[2 * iw + 2]], axis=1)
    last = jnp.stack([bnds[62], bnds[63], bnds[63], bnds[63]])[None, :]
    bnd2 = jnp.concatenate([rows4, last], axis=0)
    bnd2 = jnp.pad(bnd2, ((0, 0), (0, 12)))

    sperm = jnp.argsort(src)
    s_s = src[sperm]
    w_s = ew[sperm]
    cs = jnp.concatenate([jnp.zeros((1,), jnp.float32), jnp.cumsum(w_s)])
    ptr = jnp.searchsorted(s_s, jnp.arange(_N + 1, dtype=jnp.int32))
    deg = cs[ptr[1:]] - cs[ptr[:-1]]
    dinv1 = jnp.where(deg > 0.0, jax.lax.rsqrt(jnp.where(deg > 0.0, deg, 1.0)),
                      0.0)
    dinv = jnp.broadcast_to(dinv1[:, None], (_N, _F))

    h = x
    for (Wk, bk) in ((W1, b1), (W2, b2)):
        u = _mul(dinv, h)
        v1 = _sprop(u, packed, bnd2)
        tx1, u1 = _mid_first(v1, dinv)
        txs = [h, tx1]
        uk = u1
        for _ in range(2, Wk.shape[0]):
            vk = _sprop(uk, packed, bnd2)
            txk, uk = _mid_recur(vk, txs[-2], dinv)
            txs.append(txk)
        h = _cheb_mm(txs, Wk, bk)

    return _attention(h, batch, gw1, gb1, gw2, gb2, gw3, gb3)


# speedup vs baseline: 1.5874x; 1.5347x over previous
import functools

import jax
import jax.numpy as jnp
from jax import lax
from jax.experimental import pallas as pl
from jax.experimental.pallas import tpu as pltpu
from jax.experimental.pallas import tpu_sc as plsc

_N = 10000
_E = 160000
_F = 512
_GRAPHS = 8

_RPB = 160
_NSTRIPE = 63
_B = 64
_BLK = 1000

_MESH = plsc.VectorSubcoreMesh(core_axis_name="c", subcore_axis_name="s")


@functools.partial(
    pl.kernel,
    mesh=_MESH,
    out_type=jax.ShapeDtypeStruct((_N, _F), jnp.float32),
    scratch_types=[
        pltpu.VMEM((_RPB, _F), jnp.float32),
        pltpu.VMEM((_B, _F), jnp.float32),
        pltpu.VMEM((256,), jnp.int32),
        pltpu.VMEM((32, 16), jnp.int32),
        pltpu.SemaphoreType.DMA,
    ],
)
def _sprop(u_hbm, pk_hbm, bnd_hbm, out_hbm, acc, rows, ebuf, bv, sem):
    cid = lax.axis_index("c")
    sid = lax.axis_index("s")
    wid = sid * 2 + cid
    lanes = jax.lax.iota(jnp.int32, 16)
    z = jnp.zeros((16,), jnp.float32)

    pltpu.sync_copy(bnd_hbm, bv)
    brow = bv[wid, pl.ds(0, 16)]

    def do_stripe(kr, nrows, subsizes, e0, e1):
        row_lo = kr * _RPB

        def zr(r, _):
            for kk in range(_F // 16):
                acc[r, pl.ds(kk * 16, 16)] = z
            return 0

        lax.fori_loop(0, nrows, zr, 0)

        eb0 = lax.div(e0, _B)
        nb = lax.div(e1 + (_B - 1), _B) - eb0

        def bbody(j, _):
            base = (eb0 + j) * _B
            pltpu.sync_copy(pk_hbm.at[pl.ds(base * 3, 192)],
                            ebuf.at[pl.ds(0, 192)])
            pltpu.async_copy(u_hbm.at[ebuf.at[pl.ds(0, _B)]], rows,
                             sem).wait()
            for t in range(_B // 16):
                off = t * 16
                pos = base + off + lanes
                m = (pos >= e0) & (pos < e1)
                dv = ebuf[pl.ds(64 + off, 16)]
                wv = lax.bitcast_convert_type(ebuf[pl.ds(128 + off, 16)],
                                              jnp.float32)
                dl = jnp.clip(dv - row_lo, 0, nrows - 1)
                wv = jnp.where(m, wv, 0.0)
                ebuf[pl.ds(64 + off, 16)] = dl
                ebuf[pl.ds(128 + off, 16)] = lax.bitcast_convert_type(
                    wv, jnp.int32)

            def rbody(r, _):
                dl = ebuf[pl.ds(64 + r, 16)][0]
                w0 = lax.bitcast_convert_type(ebuf[pl.ds(128 + r, 16)],
                                              jnp.float32)[0]
                wb = jnp.broadcast_to(w0, (16,))
                for kk in range(_F // 16):
                    sl = pl.ds(kk * 16, 16)
                    acc[dl, sl] = acc[dl, sl] + wb * rows[r, sl]
                return 0

            lax.fori_loop(0, _B, rbody, 0)
            return 0

        lax.fori_loop(0, nb, bbody, 0)

        off = 0
        for sz in subsizes:
            pltpu.sync_copy(acc.at[pl.ds(off, sz)],
                            out_hbm.at[pl.ds(row_lo + off, sz)])
            off += sz

    @pl.when(wid < 31)
    def _():
        do_stripe(2 * wid, _RPB, (64, 64, 32), brow[0], brow[1])
        do_stripe(2 * wid + 1, _RPB, (64, 64, 32), brow[2], brow[3])

    @pl.when(wid == 31)
    def _():
        do_stripe(62, _N - 62 * _RPB, (64, 16), brow[0], brow[1])


def _ew_call(body, n_out, *arrays):
    n = arrays[0].shape[0]
    outs = tuple(jax.ShapeDtypeStruct((n, _F), jnp.float32)
                 for _ in range(n_out))
    out_specs = tuple(pl.BlockSpec((_BLK, _F), lambda i: (i, 0))
                      for _ in range(n_out))
    if n_out == 1:
        outs = outs[0]
        out_specs = out_specs[0]
    return pl.pallas_call(
        body,
        grid=(n // _BLK,),
        in_specs=[pl.BlockSpec((_BLK, _F), lambda i: (i, 0))] * len(arrays),
        out_specs=out_specs,
        out_shape=outs,
    )(*arrays)


def _mul(a, b):
    def body(a_ref, b_ref, o_ref):
        o_ref[...] = a_ref[...] * b_ref[...]

    return _ew_call(body, 1, a, b)


def _mid_first(v, dinv):
    def body(v_ref, d_ref, tx_ref, u_ref):
        tx = -(d_ref[...] * v_ref[...])
        tx_ref[...] = tx
        u_ref[...] = d_ref[...] * tx

    return _ew_call(body, 2, v, dinv)


def _mid_recur(v, tprev, dinv):
    def body(v_ref, t_ref, d_ref, tx_ref, u_ref):
        tx = -2.0 * (d_ref[...] * v_ref[...]) - t_ref[...]
        tx_ref[...] = tx
        u_ref[...] = d_ref[...] * tx

    return _ew_call(body, 2, v, tprev, dinv)


def _cheb_mm(txs, W, b):
    K = W.shape[0]
    n, f_in = txs[0].shape
    f_out = W.shape[2]

    def body(*refs):
        tx_refs = refs[:K]
        w_ref, b_ref, o_ref = refs[K], refs[K + 1], refs[K + 2]
        acc = jnp.dot(tx_refs[0][...], w_ref[0], preferred_element_type=jnp.float32)
        for k in range(1, K):
            acc = acc + jnp.dot(tx_refs[k][...], w_ref[k],
                                preferred_element_type=jnp.float32)
        o_ref[...] = jnp.maximum(acc + b_ref[...], 0.0)

    return pl.pallas_call(
        body,
        grid=(n // _BLK,),
        in_specs=[pl.BlockSpec((_BLK, f_in), lambda i: (i, 0))] * K
        + [
            pl.BlockSpec((K, f_in, f_out), lambda i: (0, 0, 0)),
            pl.BlockSpec((1, f_out), lambda i: (0, 0)),
        ],
        out_specs=pl.BlockSpec((_BLK, f_out), lambda i: (i, 0)),
        out_shape=jax.ShapeDtypeStruct((n, f_out), jnp.float32),
    )(*txs, W, b.reshape(1, -1))


def _attention(h, batch, gw1, gb1, gw2, gb2, gw3, gb3):
    n, f = h.shape

    def body(h_ref, b_ref, w1_ref, b1_ref, w2_ref, b2_ref, w3_ref, b3_ref,
             o_ref):
        hv = h_ref[...]
        a = jnp.maximum(jnp.dot(hv, w1_ref[...],
                                preferred_element_type=jnp.float32)
                        + b1_ref[...], 0.0)
        a = jnp.maximum(jnp.dot(a, w2_ref[...],
                                preferred_element_type=jnp.float32)
                        + b2_ref[...], 0.0)
        gate = jnp.tanh(jnp.dot(a, w3_ref[...],
                                preferred_element_type=jnp.float32)
                        + b3_ref[...])
        seg = jax.lax.broadcasted_iota(jnp.int32, (1, _GRAPHS), 1)
        onehot = (b_ref[...] == seg).astype(jnp.float32)
        neg = jnp.float32(-1e30)
        masked = jnp.where(onehot > 0, gate, neg)
        seg_max = jnp.max(masked, axis=0, keepdims=True)
        seg_max = jnp.where(seg_max > neg * 0.5, seg_max, 0.0)
        gmax = jnp.sum(onehot * seg_max, axis=1, keepdims=True)
        g = jnp.exp(gate - gmax)
        denom = jnp.sum(onehot * g, axis=0, keepdims=True)
        safe = jnp.where(denom > 0, denom, 1.0)
        dn = jnp.sum(onehot * safe, axis=1, keepdims=True)
        alpha = g / dn
        weights = onehot * alpha
        o_ref[...] = jax.lax.dot_general(
            weights, hv, (((0,), (0,)), ((), ())),
            preferred_element_type=jnp.float32)

    return pl.pallas_call(
        body,
        out_shape=jax.ShapeDtypeStruct((_GRAPHS, f), jnp.float32),
    )(h, batch.reshape(n, 1), gw1, gb1.reshape(1, -1), gw2,
      gb2.reshape(1, -1), gw3, gb3.reshape(1, -1))


def kernel(x, edge_index, edge_weight, batch, W1, b1, W2, b2,
           gw1, gb1, gw2, gb2, gw3, gb3):
    src = edge_index[0].astype(jnp.int32)
    dst = edge_index[1].astype(jnp.int32)
    ew = edge_weight.astype(jnp.float32)

    perm = jnp.argsort(dst)
    ss_ = src[perm]
    ds_ = dst[perm]
    ws_ = ew[perm]
    packed = jnp.stack(
        [ss_.reshape(-1, _B), ds_.reshape(-1, _B),
         lax.bitcast_convert_type(ws_, jnp.int32).reshape(-1, _B)],
        axis=1).reshape(-1)
    bnds = jnp.searchsorted(
        ds_, jnp.arange(0, 64 * _RPB, _RPB, dtype=jnp.int32)).astype(jnp.int32)
    iw = jnp.arange(31, dtype=jnp.int32)
    rows4 = jnp.stack([bnds[2 * iw], bnds[2 * iw + 1],
                       bnds[2 * iw + 1], bnds[2 * iw + 2]], axis=1)
    last = jnp.stack([bnds[62], bnds[63], bnds[63], bnds[63]])[None, :]
    bnd2 = jnp.concatenate([rows4, last], axis=0)
    bnd2 = jnp.pad(bnd2, ((0, 0), (0, 12)))

    deg = jax.ops.segment_sum(ew, src, num_segments=_N)
    dinv1 = jnp.where(deg > 0.0, jax.lax.rsqrt(jnp.where(deg > 0.0, deg, 1.0)),
                      0.0)
    dinv = jnp.broadcast_to(dinv1[:, None], (_N, _F))

    h = x
    for (Wk, bk) in ((W1, b1), (W2, b2)):
        u = _mul(dinv, h)
        v1 = _sprop(u, packed, bnd2)
        tx1, u1 = _mid_first(v1, dinv)
        txs = [h, tx1]
        uk = u1
        for _ in range(2, Wk.shape[0]):
            vk = _sprop(uk, packed, bnd2)
            txk, uk = _mid_recur(vk, txs[-2], dinv)
            txs.append(txk)
        h = _cheb_mm(txs, Wk, bk)

    return _attention(h, batch, gw1, gb1, gw2, gb2, gw3, gb3)
